# baseline (speedup 1.0000x reference)
import functools

import jax
import jax.numpy as jnp
from jax import lax
from jax.experimental import pallas as pl
from jax.experimental.pallas import tpu as pltpu
from jax.experimental.pallas import tpu_sc as plsc

NC = 2
NS = 16
NW = NC * NS
L = 16


def _make_kernel(B, E, R, D):
    assert B % NW == 0
    bw = B // NW
    nchunk = bw // L
    mesh = plsc.VectorSubcoreMesh(core_axis_name="c", subcore_axis_name="s")

    @functools.partial(
        pl.kernel,
        out_type=(
            jax.ShapeDtypeStruct((B,), jnp.float32),
            jax.ShapeDtypeStruct((NW, L), jnp.float32),
        ),
        mesh=mesh,
        compiler_params=pltpu.CompilerParams(
            needs_layout_passes=False, use_tc_tiling_on_sc=False),
        scratch_types=[
            pltpu.VMEM((bw,), jnp.int32),
            pltpu.VMEM((bw,), jnp.int32),
            pltpu.VMEM((bw,), jnp.int32),
            pltpu.VMEM((bw, D), jnp.float32),
            pltpu.VMEM((bw, D), jnp.float32),
            pltpu.VMEM((bw, D), jnp.float32),
            pltpu.VMEM((bw,), jnp.float32),
            pltpu.VMEM((bw,), jnp.float32),
            pltpu.VMEM((L,), jnp.float32),
            pltpu.VMEM((L,), jnp.float32),
            pltpu.VMEM((L,), jnp.float32),
            pltpu.SemaphoreType.DMA,
            pltpu.SemaphoreType.DMA,
            pltpu.SemaphoreType.DMA,
        ],
    )
    def k(h_hbm, t_hbm, r_hbm, sc_hbm, ent_hbm, rel_hbm, w_hbm, b_hbm,
          preds_hbm, part_hbm,
          hi_v, ti_v, ri_v, he_v, te_v, re_v, sc_v, pr_v, w_v, b_v, acc_v,
          sem_h, sem_t, sem_r):
        wid = lax.axis_index("s") * NC + lax.axis_index("c")
        base = wid * bw

        pltpu.sync_copy(h_hbm.at[pl.ds(base, bw)], hi_v)
        pltpu.sync_copy(t_hbm.at[pl.ds(base, bw)], ti_v)
        pltpu.sync_copy(r_hbm.at[pl.ds(base, bw)], ri_v)
        cp_h = pltpu.async_copy(ent_hbm.at[hi_v], he_v, sem_h)
        cp_t = pltpu.async_copy(ent_hbm.at[ti_v], te_v, sem_t)
        cp_r = pltpu.async_copy(rel_hbm.at[ri_v], re_v, sem_r)
        pltpu.sync_copy(sc_hbm.at[pl.ds(base, bw)], sc_v)
        pltpu.sync_copy(w_hbm, w_v)
        pltpu.sync_copy(b_hbm, b_v)
        cp_h.wait()
        cp_t.wait()
        cp_r.wait()

        acc_v[...] = jnp.zeros((L,), jnp.float32)
        iota = lax.iota(jnp.int32, L)

        def chunk(c, carry):
            rows = c * L + iota
            x = jnp.zeros((L,), jnp.float32)
            for j in range(D):
                col = jnp.full((L,), j, jnp.int32)
                hv = plsc.load_gather(he_v, [rows, col])
                tv = plsc.load_gather(te_v, [rows, col])
                rv = plsc.load_gather(re_v, [rows, col])
                x = x + hv * tv * rv
            t = w_v[...] * x + b_v[...]
            p = 1.0 / (1.0 + jnp.exp(-t))
            pr_v[pl.ds(c * L, L)] = p
            d = p - sc_v[pl.ds(c * L, L)]
            acc_v[...] = acc_v[...] + d * d
            return carry

        lax.fori_loop(0, nchunk, chunk, 0)

        pltpu.sync_copy(pr_v, preds_hbm.at[pl.ds(base, bw)])
        pltpu.sync_copy(acc_v, part_hbm.at[wid])

    return k


def kernel(h, r, t, scores, ent_emb, rel_emb, w, b):
    B = h.shape[0]
    E, D = ent_emb.shape
    R = rel_emb.shape[0]
    h32 = h.astype(jnp.int32)
    t32 = t.astype(jnp.int32)
    r32 = r.astype(jnp.int32)
    w16 = jnp.broadcast_to(w.astype(jnp.float32), (L,))
    b16 = jnp.broadcast_to(b.astype(jnp.float32), (L,))
    k = _make_kernel(B, E, R, D)
    preds, partials = k(h32, t32, r32, scores, ent_emb, rel_emb, w16, b16)
    loss = jnp.sum(partials) / B
    return (preds, loss)
